# baseline (speedup 1.0000x reference)
import functools

import jax
import jax.numpy as jnp
from jax import lax
from jax.experimental import pallas as pl
from jax.experimental.pallas import tpu as pltpu
from jax.experimental.pallas import tpu_sc as plsc

N = 10000
E = 320000
D = 128

NC = 2
NS = 16
NW = NC * NS
EPT = E // NW
K = 125
NCH = EPT // K
NP = 10240
RPT = NP // NS
DEG_W = 16

_mesh = plsc.VectorSubcoreMesh(core_axis_name="c", subcore_axis_name="s")


@functools.partial(
    pl.kernel,
    out_type=jax.ShapeDtypeStruct((NC, NP, DEG_W), jnp.float32),
    mesh=_mesh,
    scratch_types=[
        pltpu.VMEM((NCH, K), jnp.int32),
        pltpu.VMEM((K, DEG_W), jnp.float32),
        pltpu.VMEM((16, DEG_W), jnp.float32),
        pltpu.VMEM_SHARED((NP, DEG_W), jnp.float32),
        pltpu.SemaphoreType.DMA,
    ],
)
def _deg_kernel(dst_hbm, out_hbm, dst_l, ones_l, zbuf, acc_sh, sem):
    c = lax.axis_index("c")
    s = lax.axis_index("s")
    wid = c * NS + s

    zero16 = jnp.zeros((16,), jnp.float32)
    e0 = jnp.where(lax.iota(jnp.int32, 16) == 0, 1.0, 0.0)

    def fill_ones(i, carry):
        ones_l[i, :] = e0
        return carry

    lax.fori_loop(0, K, fill_ones, 0)

    def fill_z(i, carry):
        zbuf[i, :] = zero16
        return carry

    lax.fori_loop(0, 16, fill_z, 0)

    def zero_acc(i, carry):
        pltpu.async_copy(zbuf, acc_sh.at[pl.ds(s * RPT + i * 16, 16)], sem)
        return carry

    lax.fori_loop(0, RPT // 16, zero_acc, 0)
    pltpu.sync_copy(dst_hbm.at[wid], dst_l)

    def zero_drain(i, carry):
        pltpu.make_async_copy(
            zbuf, acc_sh.at[pl.ds(s * RPT + i * 16, 16)], sem).wait()
        return carry

    lax.fori_loop(0, RPT // 16, zero_drain, 0)
    plsc.subcore_barrier()

    DB = 16

    def batch(b, carry):
        def fire(j, carry2):
            pltpu.async_copy(ones_l, acc_sh.at[dst_l.at[b * DB + j]], sem,
                             add=True)
            return carry2

        lax.fori_loop(0, DB, fire, 0)

        def drain(j, carry2):
            pltpu.make_async_copy(
                ones_l, acc_sh.at[dst_l.at[b * DB + j]], sem).wait()
            return carry2

        lax.fori_loop(0, DB, drain, 0)
        return carry

    lax.fori_loop(0, NCH // DB, batch, 0)
    plsc.subcore_barrier()

    pltpu.sync_copy(acc_sh.at[pl.ds(s * RPT, RPT)],
                    out_hbm.at[c, pl.ds(s * RPT, RPT)])


@functools.partial(
    pl.kernel,
    out_type=jax.ShapeDtypeStruct((NC, NP, D), jnp.float32),
    mesh=_mesh,
    scratch_types=[
        pltpu.VMEM((NCH // 2, K), jnp.int32),
        pltpu.VMEM((NCH // 2, K), jnp.int32),
        pltpu.VMEM((K, D), jnp.float32),
        pltpu.VMEM((K, D), jnp.float32),
        pltpu.VMEM((16, D), jnp.float32),
        pltpu.VMEM_SHARED((NP, D), jnp.float32),
        pltpu.SemaphoreType.DMA,
        pltpu.SemaphoreType.DMA,
        pltpu.SemaphoreType.DMA,
        pltpu.SemaphoreType.DMA,
    ],
)
def _edge_kernel(hp_hbm, src_hbm, dst_hbm, out_hbm,
                 src_l, dst_l, rows0, rows1, zbuf, acc_sh,
                 semg0, semg1, sems0, sems1):
    c = lax.axis_index("c")
    s = lax.axis_index("s")
    wid = c * NS + s

    zero16 = jnp.zeros((16,), jnp.float32)

    def fill_z(i, carry):
        def fill_lane(q, carry2):
            zbuf[i, pl.ds(q * 16, 16)] = zero16
            return carry2
        return lax.fori_loop(0, D // 16, fill_lane, carry)

    lax.fori_loop(0, 16, fill_z, 0)

    def zero_acc(i, carry):
        pltpu.async_copy(zbuf, acc_sh.at[pl.ds(s * RPT + i * 16, 16)], semg0)
        return carry

    lax.fori_loop(0, RPT // 16, zero_acc, 0)

    def zero_drain(i, carry):
        pltpu.make_async_copy(
            zbuf, acc_sh.at[pl.ds(s * RPT + i * 16, 16)], semg0).wait()
        return carry

    lax.fori_loop(0, RPT // 16, zero_drain, 0)

    plsc.subcore_barrier()

    NH = NCH // 2
    NPAIR = NH // 2

    def half_body(h, carry):
        pltpu.sync_copy(src_hbm.at[wid, pl.ds(h * NH, NH)], src_l)
        pltpu.sync_copy(dst_hbm.at[wid, pl.ds(h * NH, NH)], dst_l)
        pltpu.async_copy(hp_hbm.at[src_l.at[0]], rows0, semg0)
        pltpu.async_copy(hp_hbm.at[src_l.at[1]], rows1, semg1)

        def body(p, carry2):
            j0 = 2 * p
            j1 = j0 + 1
            pltpu.make_async_copy(hp_hbm.at[src_l.at[j0]], rows0, semg0).wait()
            pltpu.async_copy(rows0, acc_sh.at[dst_l.at[j0]], sems0, add=True)
            pltpu.make_async_copy(hp_hbm.at[src_l.at[j1]], rows1, semg1).wait()
            pltpu.async_copy(rows1, acc_sh.at[dst_l.at[j1]], sems1, add=True)
            pltpu.make_async_copy(rows0, acc_sh.at[dst_l.at[j0]], sems0).wait()
            pltpu.async_copy(hp_hbm.at[src_l.at[j0 + 2]], rows0, semg0)
            pltpu.make_async_copy(rows1, acc_sh.at[dst_l.at[j1]], sems1).wait()
            pltpu.async_copy(hp_hbm.at[src_l.at[j1 + 2]], rows1, semg1)
            return carry2

        lax.fori_loop(0, NPAIR - 1, body, 0)

        pltpu.make_async_copy(hp_hbm.at[src_l.at[NH - 2]], rows0, semg0).wait()
        pltpu.sync_copy(rows0, acc_sh.at[dst_l.at[NH - 2]], add=True)
        pltpu.make_async_copy(hp_hbm.at[src_l.at[NH - 1]], rows1, semg1).wait()
        pltpu.sync_copy(rows1, acc_sh.at[dst_l.at[NH - 1]], add=True)
        return carry

    lax.fori_loop(0, 2, half_body, 0)
    plsc.subcore_barrier()

    pltpu.sync_copy(acc_sh.at[pl.ds(s * RPT, RPT)],
                    out_hbm.at[c, pl.ds(s * RPT, RPT)])


def _mm_body(x_ref, w_ref, degp_ref, hp_ref):
    deg = jnp.sum(degp_ref[...], axis=(0, 2)) + 1.0
    dinv = lax.rsqrt(deg)
    h = jnp.dot(x_ref[...], w_ref[...], preferred_element_type=jnp.float32)
    hp_ref[...] = h * dinv[:, None]


def _bn_body(accp_ref, hp_ref, degp_ref, gamma_ref, beta_ref, y_ref,
             pre_s, sum_s, sumsq_s, scale_s, shift_s):
    p = pl.program_id(0)
    i = pl.program_id(1)

    @pl.when(p == 0)
    def _():
        deg = jnp.sum(degp_ref[...], axis=(0, 2)) + 1.0
        dinv = lax.rsqrt(deg)
        pre = (accp_ref[0] + accp_ref[1] + hp_ref[...]) * dinv[:, None]
        pre_s[i] = pre

        @pl.when(i == 0)
        def _():
            sum_s[...] = jnp.zeros_like(sum_s)
            sumsq_s[...] = jnp.zeros_like(sumsq_s)

        sum_s[...] += jnp.sum(pre, axis=0, keepdims=True)
        sumsq_s[...] += jnp.sum(pre * pre, axis=0, keepdims=True)
        y_ref[...] = pre

    @pl.when(p == 1)
    def _():
        @pl.when(i == 0)
        def _():
            mean = sum_s[...] / N
            var = sumsq_s[...] / N - mean * mean
            rstd = lax.rsqrt(var + 1e-5)
            scale_s[...] = rstd * gamma_ref[...]
            shift_s[...] = beta_ref[...] - mean * rstd * gamma_ref[...]

        y_ref[...] = jnp.maximum(pre_s[i] * scale_s[...] + shift_s[...], 0.0)


BR = 1000


def kernel(x, edge_index, W, gamma, beta):
    src = edge_index[0].astype(jnp.int32).reshape(NW, NCH, K)
    dst = edge_index[1].astype(jnp.int32).reshape(NW, NCH, K)

    degp = _deg_kernel(dst)

    hp = pl.pallas_call(
        _mm_body,
        grid=(N // BR,),
        in_specs=[
            pl.BlockSpec((BR, D), lambda i: (i, 0)),
            pl.BlockSpec((D, D), lambda i: (0, 0)),
            pl.BlockSpec((NC, BR, DEG_W), lambda i: (0, i, 0)),
        ],
        out_specs=pl.BlockSpec((BR, D), lambda i: (i, 0)),
        out_shape=jax.ShapeDtypeStruct((N, D), jnp.float32),
    )(x, W, degp)

    accp = _edge_kernel(hp, src, dst)

    y = pl.pallas_call(
        _bn_body,
        grid=(2, N // BR),
        in_specs=[
            pl.BlockSpec((NC, BR, D), lambda p, i: (0, i * (1 - p), 0)),
            pl.BlockSpec((BR, D), lambda p, i: (i * (1 - p), 0)),
            pl.BlockSpec((NC, BR, DEG_W), lambda p, i: (0, i * (1 - p), 0)),
            pl.BlockSpec((1, D), lambda p, i: (0, 0)),
            pl.BlockSpec((1, D), lambda p, i: (0, 0)),
        ],
        out_specs=pl.BlockSpec((BR, D), lambda p, i: (i, 0)),
        out_shape=jax.ShapeDtypeStruct((N, D), jnp.float32),
        scratch_shapes=[
            pltpu.VMEM((N // BR, BR, D), jnp.float32),
            pltpu.VMEM((1, D), jnp.float32),
            pltpu.VMEM((1, D), jnp.float32),
            pltpu.VMEM((1, D), jnp.float32),
            pltpu.VMEM((1, D), jnp.float32),
        ],
    )(accp, hp, degp, gamma[None, :], beta[None, :])
    return y


# speedup vs baseline: 1.1931x; 1.1931x over previous
import functools

import jax
import jax.numpy as jnp
from jax import lax
from jax.experimental import pallas as pl
from jax.experimental.pallas import tpu as pltpu
from jax.experimental.pallas import tpu_sc as plsc

N = 10000
E = 320000
D = 128

NC = 2
NS = 16
NW = NC * NS
EPT = E // NW
K = 125
NCH = EPT // K
NP = 10240
RPT = NP // NS
DEG_W = 16

_mesh = plsc.VectorSubcoreMesh(core_axis_name="c", subcore_axis_name="s")


@functools.partial(
    pl.kernel,
    out_type=jax.ShapeDtypeStruct((NC, NP, DEG_W), jnp.float32),
    mesh=_mesh,
    scratch_types=[
        pltpu.VMEM((NCH, K), jnp.int32),
        pltpu.VMEM((K, DEG_W), jnp.float32),
        pltpu.VMEM((16, DEG_W), jnp.float32),
        pltpu.VMEM_SHARED((NP, DEG_W), jnp.float32),
        pltpu.SemaphoreType.DMA,
    ],
)
def _deg_kernel(dst_hbm, out_hbm, dst_l, ones_l, zbuf, acc_sh, sem):
    c = lax.axis_index("c")
    s = lax.axis_index("s")
    wid = c * NS + s

    zero16 = jnp.zeros((16,), jnp.float32)
    e0 = jnp.where(lax.iota(jnp.int32, 16) == 0, 1.0, 0.0)

    def fill_ones(i, carry):
        ones_l[i, :] = e0
        return carry

    lax.fori_loop(0, K, fill_ones, 0)

    def fill_z(i, carry):
        zbuf[i, :] = zero16
        return carry

    lax.fori_loop(0, 16, fill_z, 0)

    def zero_acc(i, carry):
        pltpu.async_copy(zbuf, acc_sh.at[pl.ds(s * RPT + i * 16, 16)], sem)
        return carry

    lax.fori_loop(0, RPT // 16, zero_acc, 0)
    pltpu.sync_copy(dst_hbm.at[wid], dst_l)

    def zero_drain(i, carry):
        pltpu.make_async_copy(
            zbuf, acc_sh.at[pl.ds(s * RPT + i * 16, 16)], sem).wait()
        return carry

    lax.fori_loop(0, RPT // 16, zero_drain, 0)
    plsc.subcore_barrier()

    DB = 16

    def batch(b, carry):
        def fire(j, carry2):
            pltpu.async_copy(ones_l, acc_sh.at[dst_l.at[b * DB + j]], sem,
                             add=True)
            return carry2

        lax.fori_loop(0, DB, fire, 0)

        def drain(j, carry2):
            pltpu.make_async_copy(
                ones_l, acc_sh.at[dst_l.at[b * DB + j]], sem).wait()
            return carry2

        lax.fori_loop(0, DB, drain, 0)
        return carry

    lax.fori_loop(0, NCH // DB, batch, 0)
    plsc.subcore_barrier()

    pltpu.sync_copy(acc_sh.at[pl.ds(s * RPT, RPT)],
                    out_hbm.at[c, pl.ds(s * RPT, RPT)])


@functools.partial(
    pl.kernel,
    out_type=jax.ShapeDtypeStruct((NC, NP, D), jnp.float32),
    mesh=_mesh,
    scratch_types=[
        pltpu.VMEM((NCH // 2, K), jnp.int32),
        pltpu.VMEM((NCH // 2, K), jnp.int32),
        pltpu.VMEM((K, D), jnp.float32),
        pltpu.VMEM((K, D), jnp.float32),
        pltpu.VMEM((16, D), jnp.float32),
        pltpu.VMEM_SHARED((NP, D), jnp.float32),
        pltpu.SemaphoreType.DMA,
        pltpu.SemaphoreType.DMA,
        pltpu.SemaphoreType.DMA,
        pltpu.SemaphoreType.DMA,
    ],
)
def _edge_kernel(hp_hbm, src_hbm, dst_hbm, out_hbm,
                 src_l, dst_l, rows0, rows1, zbuf, acc_sh,
                 semg0, semg1, sems0, sems1):
    c = lax.axis_index("c")
    s = lax.axis_index("s")
    wid = c * NS + s

    zero16 = jnp.zeros((16,), jnp.float32)

    def fill_z(i, carry):
        def fill_lane(q, carry2):
            zbuf[i, pl.ds(q * 16, 16)] = zero16
            return carry2
        return lax.fori_loop(0, D // 16, fill_lane, carry)

    lax.fori_loop(0, 16, fill_z, 0)

    def zero_acc(i, carry):
        pltpu.async_copy(zbuf, acc_sh.at[pl.ds(s * RPT + i * 16, 16)], semg0)
        return carry

    lax.fori_loop(0, RPT // 16, zero_acc, 0)

    def zero_drain(i, carry):
        pltpu.make_async_copy(
            zbuf, acc_sh.at[pl.ds(s * RPT + i * 16, 16)], semg0).wait()
        return carry

    lax.fori_loop(0, RPT // 16, zero_drain, 0)

    plsc.subcore_barrier()

    NH = NCH // 2
    NPAIR = NH // 2

    def half_body(h, carry):
        pltpu.sync_copy(src_hbm.at[wid, pl.ds(h * NH, NH)], src_l)
        pltpu.sync_copy(dst_hbm.at[wid, pl.ds(h * NH, NH)], dst_l)
        pltpu.async_copy(hp_hbm.at[src_l.at[0]], rows0, semg0)
        pltpu.async_copy(hp_hbm.at[src_l.at[1]], rows1, semg1)

        def body(p, carry2):
            j0 = 2 * p
            j1 = j0 + 1
            pltpu.make_async_copy(hp_hbm.at[src_l.at[j0]], rows0, semg0).wait()
            pltpu.sync_copy(rows0, acc_sh.at[dst_l.at[j0]], add=True)
            pltpu.async_copy(hp_hbm.at[src_l.at[j0 + 2]], rows0, semg0)
            pltpu.make_async_copy(hp_hbm.at[src_l.at[j1]], rows1, semg1).wait()
            pltpu.sync_copy(rows1, acc_sh.at[dst_l.at[j1]], add=True)
            pltpu.async_copy(hp_hbm.at[src_l.at[j1 + 2]], rows1, semg1)
            return carry2

        lax.fori_loop(0, NPAIR - 1, body, 0)

        pltpu.make_async_copy(hp_hbm.at[src_l.at[NH - 2]], rows0, semg0).wait()
        pltpu.sync_copy(rows0, acc_sh.at[dst_l.at[NH - 2]], add=True)
        pltpu.make_async_copy(hp_hbm.at[src_l.at[NH - 1]], rows1, semg1).wait()
        pltpu.sync_copy(rows1, acc_sh.at[dst_l.at[NH - 1]], add=True)
        return carry

    lax.fori_loop(0, 2, half_body, 0)
    plsc.subcore_barrier()

    pltpu.sync_copy(acc_sh.at[pl.ds(s * RPT, RPT)],
                    out_hbm.at[c, pl.ds(s * RPT, RPT)])


def _mm_body(x_ref, w_ref, degp_ref, hp_ref):
    deg = jnp.sum(degp_ref[...], axis=(0, 2)) + 1.0
    dinv = lax.rsqrt(deg)
    h = jnp.dot(x_ref[...], w_ref[...], preferred_element_type=jnp.float32)
    hp_ref[...] = h * dinv[:, None]


def _bn_body(accp_ref, hp_ref, degp_ref, gamma_ref, beta_ref, y_ref,
             pre_s, sum_s, sumsq_s, scale_s, shift_s):
    p = pl.program_id(0)
    i = pl.program_id(1)

    @pl.when(p == 0)
    def _():
        deg = jnp.sum(degp_ref[...], axis=(0, 2)) + 1.0
        dinv = lax.rsqrt(deg)
        pre = (accp_ref[0] + accp_ref[1] + hp_ref[...]) * dinv[:, None]
        pre_s[i] = pre

        @pl.when(i == 0)
        def _():
            sum_s[...] = jnp.zeros_like(sum_s)
            sumsq_s[...] = jnp.zeros_like(sumsq_s)

        sum_s[...] += jnp.sum(pre, axis=0, keepdims=True)
        sumsq_s[...] += jnp.sum(pre * pre, axis=0, keepdims=True)
        y_ref[...] = pre

    @pl.when(p == 1)
    def _():
        @pl.when(i == 0)
        def _():
            mean = sum_s[...] / N
            var = sumsq_s[...] / N - mean * mean
            rstd = lax.rsqrt(var + 1e-5)
            scale_s[...] = rstd * gamma_ref[...]
            shift_s[...] = beta_ref[...] - mean * rstd * gamma_ref[...]

        y_ref[...] = jnp.maximum(pre_s[i] * scale_s[...] + shift_s[...], 0.0)


BR = 1000


def kernel(x, edge_index, W, gamma, beta):
    src = edge_index[0].astype(jnp.int32).reshape(NW, NCH, K)
    dst = edge_index[1].astype(jnp.int32).reshape(NW, NCH, K)

    degp = _deg_kernel(dst)

    hp = pl.pallas_call(
        _mm_body,
        grid=(N // BR,),
        in_specs=[
            pl.BlockSpec((BR, D), lambda i: (i, 0)),
            pl.BlockSpec((D, D), lambda i: (0, 0)),
            pl.BlockSpec((NC, BR, DEG_W), lambda i: (0, i, 0)),
        ],
        out_specs=pl.BlockSpec((BR, D), lambda i: (i, 0)),
        out_shape=jax.ShapeDtypeStruct((N, D), jnp.float32),
    )(x, W, degp)

    accp = _edge_kernel(hp, src, dst)

    y = pl.pallas_call(
        _bn_body,
        grid=(2, N // BR),
        in_specs=[
            pl.BlockSpec((NC, BR, D), lambda p, i: (0, i * (1 - p), 0)),
            pl.BlockSpec((BR, D), lambda p, i: (i * (1 - p), 0)),
            pl.BlockSpec((NC, BR, DEG_W), lambda p, i: (0, i * (1 - p), 0)),
            pl.BlockSpec((1, D), lambda p, i: (0, 0)),
            pl.BlockSpec((1, D), lambda p, i: (0, 0)),
        ],
        out_specs=pl.BlockSpec((BR, D), lambda p, i: (i, 0)),
        out_shape=jax.ShapeDtypeStruct((N, D), jnp.float32),
        scratch_shapes=[
            pltpu.VMEM((N // BR, BR, D), jnp.float32),
            pltpu.VMEM((1, D), jnp.float32),
            pltpu.VMEM((1, D), jnp.float32),
            pltpu.VMEM((1, D), jnp.float32),
            pltpu.VMEM((1, D), jnp.float32),
        ],
    )(accp, hp, degp, gamma[None, :], beta[None, :])
    return y
